# baseline (speedup 1.0000x reference)
import functools

import jax
import jax.numpy as jnp
from jax import lax
from jax.experimental import pallas as pl
from jax.experimental.pallas import tpu as pltpu
from jax.experimental.pallas import tpu_sc as plsc

NC = 2
NS = 16
NW = NC * NS
CH = 80
EPS = 1e-5


def _premult(nf, We1, be1):
    N, D = nf.shape
    BN = 1000
    grid = (N // BN,)

    def body(nf_r, w1_r, b1_r, ps_r, pr_r):
        w1 = w1_r[...]
        x = nf_r[...]
        ps_r[...] = jnp.dot(x, w1[0:D], preferred_element_type=jnp.float32) + b1_r[...]
        pr_r[...] = jnp.dot(x, w1[D:2 * D], preferred_element_type=jnp.float32)

    f = pl.pallas_call(
        body,
        grid=grid,
        in_specs=[
            pl.BlockSpec((BN, D), lambda i: (i, 0)),
            pl.BlockSpec((3 * D, D), lambda i: (0, 0)),
            pl.BlockSpec((1, D), lambda i: (0, 0)),
        ],
        out_specs=[
            pl.BlockSpec((BN, D), lambda i: (i, 0)),
            pl.BlockSpec((BN, D), lambda i: (i, 0)),
        ],
        out_shape=[
            jax.ShapeDtypeStruct((N, D), jnp.float32),
            jax.ShapeDtypeStruct((N, D), jnp.float32),
        ],
    )
    return f(nf, We1, be1.reshape(1, D))


K = 5


def _sc_gather(ps, pr, snd3d, rcv3d):
    N, D = ps.shape
    _, nch, _ = snd3d.shape
    E = NW * nch * CH
    kk = K
    ng = nch // kk
    mesh = plsc.VectorSubcoreMesh(core_axis_name="c", subcore_axis_name="s")

    @functools.partial(
        pl.kernel,
        mesh=mesh,
        out_type=jax.ShapeDtypeStruct((E, D), jnp.float32),
        scratch_types=[
            pltpu.VMEM((nch, CH), jnp.int32),
            pltpu.VMEM((nch, CH), jnp.int32),
            pltpu.VMEM((kk, CH, D), jnp.float32),
            [pltpu.SemaphoreType.DMA] * kk,
            [pltpu.SemaphoreType.DMA] * kk,
        ],
    )
    def k(ps_h, pr_h, snd, rcv, g_out, idx_s, idx_r, buf, sem_g, sem_o):
        wid = lax.axis_index("s") * NC + lax.axis_index("c")
        ebase = wid * nch * CH

        pltpu.sync_copy(snd.at[wid], idx_s)
        pltpu.sync_copy(rcv.at[wid], idx_r)

        def group(gi, carry):
            for b in range(kk):
                i = gi * kk + b
                pltpu.make_async_copy(
                    ps_h.at[idx_s.at[i]], buf.at[b], sem_g[b]).start()
            for b in range(kk):
                i = gi * kk + b
                pltpu.make_async_copy(
                    ps_h.at[idx_s.at[i]], buf.at[b], sem_g[b]).wait()
                pltpu.make_async_copy(
                    pr_h.at[idx_r.at[i]], buf.at[b], sem_g[b]).start(add=True)
            for b in range(kk):
                i = gi * kk + b
                pltpu.make_async_copy(
                    pr_h.at[idx_r.at[i]], buf.at[b], sem_g[b]).wait()
                pltpu.make_async_copy(
                    buf.at[b], g_out.at[pl.ds(ebase + i * CH, CH)],
                    sem_o[b]).start()
            for b in range(kk):
                i = gi * kk + b
                pltpu.make_async_copy(
                    buf.at[b], g_out.at[pl.ds(ebase + i * CH, CH)],
                    sem_o[b]).wait()
            return carry

        lax.fori_loop(0, ng, group, 0)

    return k(ps, pr, snd3d, rcv3d)


KS = 3


def _sc_scatter(updated_edges, rcv3d, N, chunks_full, chunk_off, init):
    E, D = updated_edges.shape
    _, nch, _ = rcv3d.shape
    ng = nch // KS
    rem = nch - ng * KS
    rpt = ((N + NS - 1) // NS + 15) // 16 * 16
    n_pad = NS * rpt
    rpt_b = rpt // 16
    mesh = plsc.VectorSubcoreMesh(core_axis_name="c", subcore_axis_name="s")

    @functools.partial(
        pl.kernel,
        mesh=mesh,
        out_type=jax.ShapeDtypeStruct((NC * n_pad, D), jnp.float32),
        scratch_types=[
            pltpu.VMEM((nch, CH), jnp.int32),
            pltpu.VMEM((KS, CH, D), jnp.float32),
            pltpu.VMEM((16, D), jnp.float32),
            pltpu.VMEM_SHARED((n_pad, D), jnp.float32),
            [pltpu.SemaphoreType.DMA] * KS,
            [pltpu.SemaphoreType.DMA] * KS,
        ],
    )
    def k(upd, rcv, *rest):
        if init is None:
            out, idx_r, rows, zbuf, agg, sem_r, sem_s = rest
        else:
            prev, out, idx_r, rows, zbuf, agg, sem_r, sem_s = rest
        cid = lax.axis_index("c")
        sid = lax.axis_index("s")
        wid = sid * NC + cid
        ebase = (wid * chunks_full + chunk_off) * CH

        pltpu.sync_copy(rcv.at[wid], idx_r)

        if init is None:
            zv = jnp.zeros((16,), jnp.float32)
            for i in range(16):
                for j in range(D // 16):
                    zbuf[i, pl.ds(j * 16, 16)] = zv

            def zstep(b, carry):
                pltpu.make_async_copy(
                    zbuf, agg.at[pl.ds(sid * rpt + b * 16, 16)],
                    sem_s[0]).start()
                return carry

            lax.fori_loop(0, rpt_b, zstep, 0)

            def zdrain(b, carry):
                pltpu.make_async_copy(
                    zbuf, agg.at[pl.ds(sid * rpt + b * 16, 16)],
                    sem_s[0]).wait()
                return carry

            lax.fori_loop(0, rpt_b, zdrain, 0)
        else:
            pltpu.sync_copy(
                prev.at[pl.ds(cid * n_pad + sid * rpt, rpt)],
                agg.at[pl.ds(sid * rpt, rpt)],
            )
        plsc.subcore_barrier()

        def group(gi, carry):
            for b in range(KS):
                i = gi * KS + b
                pltpu.make_async_copy(
                    upd.at[pl.ds(ebase + i * CH, CH)], rows.at[b],
                    sem_r[b]).start()
            for b in range(KS):
                i = gi * KS + b
                pltpu.make_async_copy(
                    upd.at[pl.ds(ebase + i * CH, CH)], rows.at[b],
                    sem_r[b]).wait()
                pltpu.make_async_copy(
                    rows.at[b], agg.at[idx_r.at[i]], sem_s[b]).start(add=True)
            for b in range(KS):
                i = gi * KS + b
                pltpu.make_async_copy(
                    rows.at[b], agg.at[idx_r.at[i]], sem_s[b]).wait()
            return carry

        lax.fori_loop(0, ng, group, 0)
        for b in range(rem):
            i = ng * KS + b
            pltpu.make_async_copy(
                upd.at[pl.ds(ebase + i * CH, CH)], rows.at[b],
                sem_r[b]).start()
        for b in range(rem):
            i = ng * KS + b
            pltpu.make_async_copy(
                upd.at[pl.ds(ebase + i * CH, CH)], rows.at[b],
                sem_r[b]).wait()
            pltpu.make_async_copy(
                rows.at[b], agg.at[idx_r.at[i]], sem_s[b]).start(add=True)
        for b in range(rem):
            i = ng * KS + b
            pltpu.make_async_copy(
                rows.at[b], agg.at[idx_r.at[i]], sem_s[b]).wait()
        plsc.subcore_barrier()

        pltpu.sync_copy(
            agg.at[pl.ds(sid * rpt, rpt)],
            out.at[pl.ds(cid * n_pad + sid * rpt, rpt)],
        )

    if init is None:
        return k(updated_edges, rcv3d), n_pad
    return k(updated_edges, rcv3d, init), n_pad


def _edge_mlp_part(g_p, ef, We1e, We2, be2, eg, eb, off_blk, s_p, bpw,
                   upd_in, new_in):
    E, D = ef.shape
    BE = 2000
    nblk = g_p.shape[0] // BE
    assert nblk == NW * s_p

    def compute(g_blk, e_blk, w1, w2, b2, gm, bt):
        h = g_blk + jnp.dot(e_blk, w1, preferred_element_type=jnp.float32)
        h = jnp.maximum(h, 0.0)
        u = jnp.dot(h, w2, preferred_element_type=jnp.float32) + b2
        mu = jnp.mean(u, axis=-1, keepdims=True)
        var = jnp.mean((u - mu) ** 2, axis=-1, keepdims=True)
        return (u - mu) * jax.lax.rsqrt(var + EPS) * gm + bt

    part_map = lambda i, s=s_p, o=off_blk: (i // s * bpw + o + i % s, 0)
    own_map = lambda i: (i, 0)
    fixed_map = lambda i: (0, 0)
    small_specs = [
        pl.BlockSpec((D, D), fixed_map),
        pl.BlockSpec((D, D), fixed_map),
        pl.BlockSpec((1, D), fixed_map),
        pl.BlockSpec((1, D), fixed_map),
        pl.BlockSpec((1, D), fixed_map),
    ]
    n_alias = (upd_in is not None) + (new_in is not None)

    def body(*refs):
        g_r, ef_r, w1_r, w2_r, b2_r, gm_r, bt_r, upd_r, new_r = refs[n_alias:]
        e = ef_r[...]
        ln = compute(g_r[...], e, w1_r[...], w2_r[...], b2_r[...],
                     gm_r[...], bt_r[...])
        upd_r[...] = ln
        new_r[...] = ln + e

    alias_args, aliases = [], {}
    if upd_in is not None:
        aliases[len(alias_args)] = 0
        alias_args.append(upd_in)
    if new_in is not None:
        aliases[len(alias_args)] = 1
        alias_args.append(new_in)

    f = pl.pallas_call(
        body,
        grid=(nblk,),
        in_specs=[pl.BlockSpec(memory_space=pl.ANY)] * n_alias
                 + [pl.BlockSpec((BE, D), own_map),
                    pl.BlockSpec((BE, D), part_map)] + small_specs,
        out_specs=[pl.BlockSpec((BE, D), part_map),
                   pl.BlockSpec((BE, D), part_map)],
        out_shape=[
            jax.ShapeDtypeStruct((E, D), jnp.float32),
            jax.ShapeDtypeStruct((E, D), jnp.float32),
        ],
        input_output_aliases=aliases,
    )
    return f(*alias_args, g_p, ef, We1e, We2, be2.reshape(1, D),
             eg.reshape(1, D), eb.reshape(1, D))


def _node_mlp(nf, p0, p1, Wn1, bn1, Wn2, bn2, ng, nb):
    N, D = nf.shape
    BN = 1000
    grid = (N // BN,)

    def body(nf_r, p0_r, p1_r, w1_r, b1_r, w2_r, b2_r, g_r, b_r, out_r):
        w1 = w1_r[...]
        x = nf_r[...]
        agg = p0_r[...] + p1_r[...]
        h = jnp.dot(x, w1[0:D], preferred_element_type=jnp.float32)
        h += jnp.dot(agg, w1[D:2 * D], preferred_element_type=jnp.float32)
        h = jnp.maximum(h + b1_r[...], 0.0)
        u = jnp.dot(h, w2_r[...], preferred_element_type=jnp.float32) + b2_r[...]
        mu = jnp.mean(u, axis=-1, keepdims=True)
        var = jnp.mean((u - mu) ** 2, axis=-1, keepdims=True)
        ln = (u - mu) * jax.lax.rsqrt(var + EPS) * g_r[...] + b_r[...]
        out_r[...] = ln + x

    f = pl.pallas_call(
        body,
        grid=grid,
        in_specs=[
            pl.BlockSpec((BN, D), lambda i: (i, 0)),
            pl.BlockSpec((BN, D), lambda i: (i, 0)),
            pl.BlockSpec((BN, D), lambda i: (i, 0)),
            pl.BlockSpec((2 * D, D), lambda i: (0, 0)),
            pl.BlockSpec((1, D), lambda i: (0, 0)),
            pl.BlockSpec((D, D), lambda i: (0, 0)),
            pl.BlockSpec((1, D), lambda i: (0, 0)),
            pl.BlockSpec((1, D), lambda i: (0, 0)),
            pl.BlockSpec((1, D), lambda i: (0, 0)),
        ],
        out_specs=pl.BlockSpec((BN, D), lambda i: (i, 0)),
        out_shape=jax.ShapeDtypeStruct((N, D), jnp.float32),
    )
    return f(nf, p0, p1, Wn1, bn1.reshape(1, D), Wn2, bn2.reshape(1, D),
             ng.reshape(1, D), nb.reshape(1, D))


def kernel(node_features, edge_features, senders, receivers,
           We1, be1, We2, be2, eg, eb,
           Wn1, bn1, Wn2, bn2, ng, nb):
    N, D = node_features.shape
    E = edge_features.shape[0]
    chunks_full = E // (NW * CH)
    bpw = chunks_full * CH // 2000
    cpb = chunks_full // bpw
    snd3 = senders.reshape(NW, chunks_full, CH)
    rcv3 = receivers.reshape(NW, chunks_full, CH)
    ps, pr = _premult(node_features, We1, be1)
    We1e = We1[2 * D:3 * D]

    sizes, offs = (1, 2, 2), (0, 1, 3)
    g0 = _sc_gather(ps, pr, snd3[:, :cpb], rcv3[:, :cpb])
    g1 = _sc_gather(ps, pr, snd3[:, cpb:3 * cpb], rcv3[:, cpb:3 * cpb])
    g2 = _sc_gather(ps, pr, snd3[:, 3 * cpb:], rcv3[:, 3 * cpb:])
    upd01, new01 = _edge_mlp_part(g0, edge_features, We1e, We2, be2, eg, eb,
                                  offs[0], sizes[0], bpw, None, None)
    upd01, new012 = _edge_mlp_part(g1, edge_features, We1e, We2, be2, eg, eb,
                                   offs[1], sizes[1], bpw, upd01, new01)
    upd2, new_edges = _edge_mlp_part(g2, edge_features, We1e, We2, be2, eg, eb,
                                     offs[2], sizes[2], bpw, None, new012)
    parts_a, n_pad = _sc_scatter(upd01, rcv3[:, :3 * cpb], N,
                                 chunks_full, 0, None)
    parts_b, _ = _sc_scatter(upd2, rcv3[:, 3 * cpb:], N,
                             chunks_full, 3 * cpb, parts_a)
    new_nodes = _node_mlp(node_features, parts_b[:N],
                          parts_b[n_pad:n_pad + N],
                          Wn1, bn1, Wn2, bn2, ng, nb)
    return (new_nodes, new_edges)


# speedup vs baseline: 1.0452x; 1.0452x over previous
import functools

import jax
import jax.numpy as jnp
from jax import lax
from jax.experimental import pallas as pl
from jax.experimental.pallas import tpu as pltpu
from jax.experimental.pallas import tpu_sc as plsc

NC = 2
NS = 16
NW = NC * NS
CH = 80
EPS = 1e-5


def _premult(nf, We1, be1):
    N, D = nf.shape
    BN = 1000
    grid = (N // BN,)

    def body(nf_r, w1_r, b1_r, ps_r, pr_r):
        w1 = w1_r[...]
        x = nf_r[...]
        ps_r[...] = jnp.dot(x, w1[0:D], preferred_element_type=jnp.float32) + b1_r[...]
        pr_r[...] = jnp.dot(x, w1[D:2 * D], preferred_element_type=jnp.float32)

    f = pl.pallas_call(
        body,
        grid=grid,
        in_specs=[
            pl.BlockSpec((BN, D), lambda i: (i, 0)),
            pl.BlockSpec((3 * D, D), lambda i: (0, 0)),
            pl.BlockSpec((1, D), lambda i: (0, 0)),
        ],
        out_specs=[
            pl.BlockSpec((BN, D), lambda i: (i, 0)),
            pl.BlockSpec((BN, D), lambda i: (i, 0)),
        ],
        out_shape=[
            jax.ShapeDtypeStruct((N, D), jnp.float32),
            jax.ShapeDtypeStruct((N, D), jnp.float32),
        ],
    )
    return f(nf, We1, be1.reshape(1, D))


K = 5


def _sc_gather(ps, pr, snd3d, rcv3d):
    N, D = ps.shape
    _, nch, _ = snd3d.shape
    E = NW * nch * CH
    kk = 3
    ng = nch // kk
    rem = nch - ng * kk
    n_stage = (N + NS * 8 - 1) // (NS * 8) * 8
    mesh = plsc.VectorSubcoreMesh(core_axis_name="c", subcore_axis_name="s")

    @functools.partial(
        pl.kernel,
        mesh=mesh,
        out_type=jax.ShapeDtypeStruct((E, D), jnp.float32),
        scratch_types=[
            pltpu.VMEM((nch, CH), jnp.int32),
            pltpu.VMEM((nch, CH), jnp.int32),
            pltpu.VMEM((kk, CH, D), jnp.float32),
            pltpu.VMEM_SHARED((NS * n_stage, D), jnp.float32),
            [pltpu.SemaphoreType.DMA] * kk,
            [pltpu.SemaphoreType.DMA] * kk,
        ],
    )
    def k(ps_h, pr_h, snd, rcv, g_out, idx_s, idx_r, buf, ps_spm, sem_g, sem_o):
        sid = lax.axis_index("s")
        wid = sid * NC + lax.axis_index("c")
        ebase = wid * nch * CH

        last = N - (NS - 1) * n_stage

        @pl.when(sid < NS - 1)
        def _():
            pltpu.sync_copy(ps_h.at[pl.ds(sid * n_stage, n_stage)],
                            ps_spm.at[pl.ds(sid * n_stage, n_stage)])

        @pl.when(sid == NS - 1)
        def _():
            pltpu.sync_copy(ps_h.at[pl.ds((NS - 1) * n_stage, last)],
                            ps_spm.at[pl.ds((NS - 1) * n_stage, last)])

        pltpu.sync_copy(snd.at[wid], idx_s)
        pltpu.sync_copy(rcv.at[wid], idx_r)
        plsc.subcore_barrier()

        def do_chunks(istart, nslots):
            for b in range(nslots):
                i = istart + b
                pltpu.make_async_copy(
                    pr_h.at[idx_r.at[i]], buf.at[b], sem_g[b]).start()
            for b in range(nslots):
                i = istart + b
                pltpu.make_async_copy(
                    pr_h.at[idx_r.at[i]], buf.at[b], sem_g[b]).wait()
                pltpu.make_async_copy(
                    ps_spm.at[idx_s.at[i]], buf.at[b], sem_g[b]).start(add=True)
            for b in range(nslots):
                i = istart + b
                pltpu.make_async_copy(
                    ps_spm.at[idx_s.at[i]], buf.at[b], sem_g[b]).wait()
                pltpu.make_async_copy(
                    buf.at[b], g_out.at[pl.ds(ebase + i * CH, CH)],
                    sem_o[b]).start()
            for b in range(nslots):
                i = istart + b
                pltpu.make_async_copy(
                    buf.at[b], g_out.at[pl.ds(ebase + i * CH, CH)],
                    sem_o[b]).wait()

        def group(gi, carry):
            do_chunks(gi * kk, kk)
            return carry

        lax.fori_loop(0, ng, group, 0)
        if rem:
            do_chunks(ng * kk, rem)

    return k(ps, pr, snd3d, rcv3d)


KS = 3


def _sc_scatter(updated_edges, rcv3d, N, chunks_full, chunk_off, init):
    E, D = updated_edges.shape
    _, nch, _ = rcv3d.shape
    ng = nch // KS
    rem = nch - ng * KS
    rpt = ((N + NS - 1) // NS + 15) // 16 * 16
    n_pad = NS * rpt
    rpt_b = rpt // 16
    mesh = plsc.VectorSubcoreMesh(core_axis_name="c", subcore_axis_name="s")

    @functools.partial(
        pl.kernel,
        mesh=mesh,
        out_type=jax.ShapeDtypeStruct((NC * n_pad, D), jnp.float32),
        scratch_types=[
            pltpu.VMEM((nch, CH), jnp.int32),
            pltpu.VMEM((KS, CH, D), jnp.float32),
            pltpu.VMEM((16, D), jnp.float32),
            pltpu.VMEM_SHARED((n_pad, D), jnp.float32),
            [pltpu.SemaphoreType.DMA] * KS,
            [pltpu.SemaphoreType.DMA] * KS,
        ],
    )
    def k(upd, rcv, *rest):
        if init is None:
            out, idx_r, rows, zbuf, agg, sem_r, sem_s = rest
        else:
            prev, out, idx_r, rows, zbuf, agg, sem_r, sem_s = rest
        cid = lax.axis_index("c")
        sid = lax.axis_index("s")
        wid = sid * NC + cid
        ebase = (wid * chunks_full + chunk_off) * CH

        pltpu.sync_copy(rcv.at[wid], idx_r)

        if init is None:
            zv = jnp.zeros((16,), jnp.float32)
            for i in range(16):
                for j in range(D // 16):
                    zbuf[i, pl.ds(j * 16, 16)] = zv

            def zstep(b, carry):
                pltpu.make_async_copy(
                    zbuf, agg.at[pl.ds(sid * rpt + b * 16, 16)],
                    sem_s[0]).start()
                return carry

            lax.fori_loop(0, rpt_b, zstep, 0)

            def zdrain(b, carry):
                pltpu.make_async_copy(
                    zbuf, agg.at[pl.ds(sid * rpt + b * 16, 16)],
                    sem_s[0]).wait()
                return carry

            lax.fori_loop(0, rpt_b, zdrain, 0)
        else:
            pltpu.sync_copy(
                prev.at[pl.ds(cid * n_pad + sid * rpt, rpt)],
                agg.at[pl.ds(sid * rpt, rpt)],
            )
        plsc.subcore_barrier()

        def group(gi, carry):
            for b in range(KS):
                i = gi * KS + b
                pltpu.make_async_copy(
                    upd.at[pl.ds(ebase + i * CH, CH)], rows.at[b],
                    sem_r[b]).start()
            for b in range(KS):
                i = gi * KS + b
                pltpu.make_async_copy(
                    upd.at[pl.ds(ebase + i * CH, CH)], rows.at[b],
                    sem_r[b]).wait()
                pltpu.make_async_copy(
                    rows.at[b], agg.at[idx_r.at[i]], sem_s[b]).start(add=True)
            for b in range(KS):
                i = gi * KS + b
                pltpu.make_async_copy(
                    rows.at[b], agg.at[idx_r.at[i]], sem_s[b]).wait()
            return carry

        lax.fori_loop(0, ng, group, 0)
        for b in range(rem):
            i = ng * KS + b
            pltpu.make_async_copy(
                upd.at[pl.ds(ebase + i * CH, CH)], rows.at[b],
                sem_r[b]).start()
        for b in range(rem):
            i = ng * KS + b
            pltpu.make_async_copy(
                upd.at[pl.ds(ebase + i * CH, CH)], rows.at[b],
                sem_r[b]).wait()
            pltpu.make_async_copy(
                rows.at[b], agg.at[idx_r.at[i]], sem_s[b]).start(add=True)
        for b in range(rem):
            i = ng * KS + b
            pltpu.make_async_copy(
                rows.at[b], agg.at[idx_r.at[i]], sem_s[b]).wait()
        plsc.subcore_barrier()

        pltpu.sync_copy(
            agg.at[pl.ds(sid * rpt, rpt)],
            out.at[pl.ds(cid * n_pad + sid * rpt, rpt)],
        )

    if init is None:
        return k(updated_edges, rcv3d), n_pad
    return k(updated_edges, rcv3d, init), n_pad


def _edge_mlp_part(g_p, ef, We1e, We2, be2, eg, eb, off_blk, s_p, bpw,
                   upd_in, new_in):
    E, D = ef.shape
    BE = 2000
    nblk = g_p.shape[0] // BE
    assert nblk == NW * s_p

    def compute(g_blk, e_blk, w1, w2, b2, gm, bt):
        h = g_blk + jnp.dot(e_blk, w1, preferred_element_type=jnp.float32)
        h = jnp.maximum(h, 0.0)
        u = jnp.dot(h, w2, preferred_element_type=jnp.float32) + b2
        mu = jnp.mean(u, axis=-1, keepdims=True)
        var = jnp.mean((u - mu) ** 2, axis=-1, keepdims=True)
        return (u - mu) * jax.lax.rsqrt(var + EPS) * gm + bt

    part_map = lambda i, s=s_p, o=off_blk: (i // s * bpw + o + i % s, 0)
    own_map = lambda i: (i, 0)
    fixed_map = lambda i: (0, 0)
    small_specs = [
        pl.BlockSpec((D, D), fixed_map),
        pl.BlockSpec((D, D), fixed_map),
        pl.BlockSpec((1, D), fixed_map),
        pl.BlockSpec((1, D), fixed_map),
        pl.BlockSpec((1, D), fixed_map),
    ]
    n_alias = (upd_in is not None) + (new_in is not None)

    def body(*refs):
        g_r, ef_r, w1_r, w2_r, b2_r, gm_r, bt_r, upd_r, new_r = refs[n_alias:]
        e = ef_r[...]
        ln = compute(g_r[...], e, w1_r[...], w2_r[...], b2_r[...],
                     gm_r[...], bt_r[...])
        upd_r[...] = ln
        new_r[...] = ln + e

    alias_args, aliases = [], {}
    if upd_in is not None:
        aliases[len(alias_args)] = 0
        alias_args.append(upd_in)
    if new_in is not None:
        aliases[len(alias_args)] = 1
        alias_args.append(new_in)

    f = pl.pallas_call(
        body,
        grid=(nblk,),
        in_specs=[pl.BlockSpec(memory_space=pl.ANY)] * n_alias
                 + [pl.BlockSpec((BE, D), own_map),
                    pl.BlockSpec((BE, D), part_map)] + small_specs,
        out_specs=[pl.BlockSpec((BE, D), part_map),
                   pl.BlockSpec((BE, D), part_map)],
        out_shape=[
            jax.ShapeDtypeStruct((E, D), jnp.float32),
            jax.ShapeDtypeStruct((E, D), jnp.float32),
        ],
        input_output_aliases=aliases,
    )
    return f(*alias_args, g_p, ef, We1e, We2, be2.reshape(1, D),
             eg.reshape(1, D), eb.reshape(1, D))


def _node_mlp(nf, p0, p1, Wn1, bn1, Wn2, bn2, ng, nb):
    N, D = nf.shape
    BN = 1000
    grid = (N // BN,)

    def body(nf_r, p0_r, p1_r, w1_r, b1_r, w2_r, b2_r, g_r, b_r, out_r):
        w1 = w1_r[...]
        x = nf_r[...]
        agg = p0_r[...] + p1_r[...]
        h = jnp.dot(x, w1[0:D], preferred_element_type=jnp.float32)
        h += jnp.dot(agg, w1[D:2 * D], preferred_element_type=jnp.float32)
        h = jnp.maximum(h + b1_r[...], 0.0)
        u = jnp.dot(h, w2_r[...], preferred_element_type=jnp.float32) + b2_r[...]
        mu = jnp.mean(u, axis=-1, keepdims=True)
        var = jnp.mean((u - mu) ** 2, axis=-1, keepdims=True)
        ln = (u - mu) * jax.lax.rsqrt(var + EPS) * g_r[...] + b_r[...]
        out_r[...] = ln + x

    f = pl.pallas_call(
        body,
        grid=grid,
        in_specs=[
            pl.BlockSpec((BN, D), lambda i: (i, 0)),
            pl.BlockSpec((BN, D), lambda i: (i, 0)),
            pl.BlockSpec((BN, D), lambda i: (i, 0)),
            pl.BlockSpec((2 * D, D), lambda i: (0, 0)),
            pl.BlockSpec((1, D), lambda i: (0, 0)),
            pl.BlockSpec((D, D), lambda i: (0, 0)),
            pl.BlockSpec((1, D), lambda i: (0, 0)),
            pl.BlockSpec((1, D), lambda i: (0, 0)),
            pl.BlockSpec((1, D), lambda i: (0, 0)),
        ],
        out_specs=pl.BlockSpec((BN, D), lambda i: (i, 0)),
        out_shape=jax.ShapeDtypeStruct((N, D), jnp.float32),
    )
    return f(nf, p0, p1, Wn1, bn1.reshape(1, D), Wn2, bn2.reshape(1, D),
             ng.reshape(1, D), nb.reshape(1, D))


def kernel(node_features, edge_features, senders, receivers,
           We1, be1, We2, be2, eg, eb,
           Wn1, bn1, Wn2, bn2, ng, nb):
    N, D = node_features.shape
    E = edge_features.shape[0]
    chunks_full = E // (NW * CH)
    bpw = chunks_full * CH // 2000
    cpb = chunks_full // bpw
    snd3 = senders.reshape(NW, chunks_full, CH)
    rcv3 = receivers.reshape(NW, chunks_full, CH)
    ps, pr = _premult(node_features, We1, be1)
    We1e = We1[2 * D:3 * D]

    sizes, offs = (1, 2, 2), (0, 1, 3)
    g0 = _sc_gather(ps, pr, snd3[:, :cpb], rcv3[:, :cpb])
    g1 = _sc_gather(ps, pr, snd3[:, cpb:3 * cpb], rcv3[:, cpb:3 * cpb])
    g2 = _sc_gather(ps, pr, snd3[:, 3 * cpb:], rcv3[:, 3 * cpb:])
    upd01, new01 = _edge_mlp_part(g0, edge_features, We1e, We2, be2, eg, eb,
                                  offs[0], sizes[0], bpw, None, None)
    upd01, new012 = _edge_mlp_part(g1, edge_features, We1e, We2, be2, eg, eb,
                                   offs[1], sizes[1], bpw, upd01, new01)
    upd2, new_edges = _edge_mlp_part(g2, edge_features, We1e, We2, be2, eg, eb,
                                     offs[2], sizes[2], bpw, None, new012)
    parts_a, n_pad = _sc_scatter(upd01, rcv3[:, :3 * cpb], N,
                                 chunks_full, 0, None)
    parts_b, _ = _sc_scatter(upd2, rcv3[:, 3 * cpb:], N,
                             chunks_full, 3 * cpb, parts_a)
    new_nodes = _node_mlp(node_features, parts_b[:N],
                          parts_b[n_pad:n_pad + N],
                          Wn1, bn1, Wn2, bn2, ng, nb)
    return (new_nodes, new_edges)


# speedup vs baseline: 1.0457x; 1.0005x over previous
import functools

import jax
import jax.numpy as jnp
from jax import lax
from jax.experimental import pallas as pl
from jax.experimental.pallas import tpu as pltpu
from jax.experimental.pallas import tpu_sc as plsc

NC = 2
NS = 16
NW = NC * NS
CH = 80
EPS = 1e-5


def _premult(nf, We1, be1):
    N, D = nf.shape
    BN = 1000
    grid = (N // BN,)

    def body(nf_r, w1_r, b1_r, ps_r, pr_r):
        w1 = w1_r[...]
        x = nf_r[...]
        ps_r[...] = jnp.dot(x, w1[0:D], preferred_element_type=jnp.float32) + b1_r[...]
        pr_r[...] = jnp.dot(x, w1[D:2 * D], preferred_element_type=jnp.float32)

    f = pl.pallas_call(
        body,
        grid=grid,
        in_specs=[
            pl.BlockSpec((BN, D), lambda i: (i, 0)),
            pl.BlockSpec((3 * D, D), lambda i: (0, 0)),
            pl.BlockSpec((1, D), lambda i: (0, 0)),
        ],
        out_specs=[
            pl.BlockSpec((BN, D), lambda i: (i, 0)),
            pl.BlockSpec((BN, D), lambda i: (i, 0)),
        ],
        out_shape=[
            jax.ShapeDtypeStruct((N, D), jnp.float32),
            jax.ShapeDtypeStruct((N, D), jnp.float32),
        ],
    )
    return f(nf, We1, be1.reshape(1, D))


K = 5


def _sc_gather(ps, pr, snd3d, rcv3d):
    N, D = ps.shape
    _, nch, _ = snd3d.shape
    E = NW * nch * CH
    kk = 3
    ng = nch // kk
    rem = nch - ng * kk
    n_stage = (N + NS * 8 - 1) // (NS * 8) * 8
    mesh = plsc.VectorSubcoreMesh(core_axis_name="c", subcore_axis_name="s")

    @functools.partial(
        pl.kernel,
        mesh=mesh,
        out_type=jax.ShapeDtypeStruct((E, D), jnp.float32),
        scratch_types=[
            pltpu.VMEM((nch, CH), jnp.int32),
            pltpu.VMEM((nch, CH), jnp.int32),
            pltpu.VMEM((kk, CH, D), jnp.float32),
            pltpu.VMEM_SHARED((NS * n_stage, D), jnp.float32),
            [pltpu.SemaphoreType.DMA] * kk,
            [pltpu.SemaphoreType.DMA] * kk,
        ],
    )
    def k(ps_h, pr_h, snd, rcv, g_out, idx_s, idx_r, buf, ps_spm, sem_g, sem_o):
        sid = lax.axis_index("s")
        wid = sid * NC + lax.axis_index("c")
        ebase = wid * nch * CH

        last = N - (NS - 1) * n_stage

        @pl.when(sid < NS - 1)
        def _():
            pltpu.sync_copy(ps_h.at[pl.ds(sid * n_stage, n_stage)],
                            ps_spm.at[pl.ds(sid * n_stage, n_stage)])

        @pl.when(sid == NS - 1)
        def _():
            pltpu.sync_copy(ps_h.at[pl.ds((NS - 1) * n_stage, last)],
                            ps_spm.at[pl.ds((NS - 1) * n_stage, last)])

        pltpu.sync_copy(snd.at[wid], idx_s)
        pltpu.sync_copy(rcv.at[wid], idx_r)
        plsc.subcore_barrier()

        def do_chunks(istart, nslots):
            for b in range(nslots):
                i = istart + b
                pltpu.make_async_copy(
                    pr_h.at[idx_r.at[i]], buf.at[b], sem_g[b]).start()
            for b in range(nslots):
                i = istart + b
                pltpu.make_async_copy(
                    pr_h.at[idx_r.at[i]], buf.at[b], sem_g[b]).wait()
                pltpu.make_async_copy(
                    ps_spm.at[idx_s.at[i]], buf.at[b], sem_g[b]).start(add=True)
            for b in range(nslots):
                i = istart + b
                pltpu.make_async_copy(
                    ps_spm.at[idx_s.at[i]], buf.at[b], sem_g[b]).wait()
                pltpu.make_async_copy(
                    buf.at[b], g_out.at[pl.ds(ebase + i * CH, CH)],
                    sem_o[b]).start()
            for b in range(nslots):
                i = istart + b
                pltpu.make_async_copy(
                    buf.at[b], g_out.at[pl.ds(ebase + i * CH, CH)],
                    sem_o[b]).wait()

        def group(gi, carry):
            do_chunks(gi * kk, kk)
            return carry

        lax.fori_loop(0, ng, group, 0)
        if rem:
            do_chunks(ng * kk, rem)

    return k(ps, pr, snd3d, rcv3d)


KS = 3


def _sc_scatter(updated_edges, rcv3d, N, chunks_full, chunk_off, init):
    E, D = updated_edges.shape
    _, nch, _ = rcv3d.shape
    ng = nch // KS
    rem = nch - ng * KS
    rpt = ((N + NS - 1) // NS + 15) // 16 * 16
    n_pad = NS * rpt
    rpt_b = rpt // 16
    mesh = plsc.VectorSubcoreMesh(core_axis_name="c", subcore_axis_name="s")

    @functools.partial(
        pl.kernel,
        mesh=mesh,
        out_type=jax.ShapeDtypeStruct((NC * n_pad, D), jnp.float32),
        scratch_types=[
            pltpu.VMEM((nch, CH), jnp.int32),
            pltpu.VMEM((KS, CH, D), jnp.float32),
            pltpu.VMEM((16, D), jnp.float32),
            pltpu.VMEM_SHARED((n_pad, D), jnp.float32),
            [pltpu.SemaphoreType.DMA] * KS,
            [pltpu.SemaphoreType.DMA] * KS,
        ],
    )
    def k(upd, rcv, *rest):
        if init is None:
            out, idx_r, rows, zbuf, agg, sem_r, sem_s = rest
        else:
            prev, out, idx_r, rows, zbuf, agg, sem_r, sem_s = rest
        cid = lax.axis_index("c")
        sid = lax.axis_index("s")
        wid = sid * NC + cid
        ebase = (wid * chunks_full + chunk_off) * CH

        pltpu.sync_copy(rcv.at[wid], idx_r)

        if init is None:
            zv = jnp.zeros((16,), jnp.float32)
            for i in range(16):
                for j in range(D // 16):
                    zbuf[i, pl.ds(j * 16, 16)] = zv

            def zstep(b, carry):
                pltpu.make_async_copy(
                    zbuf, agg.at[pl.ds(sid * rpt + b * 16, 16)],
                    sem_s[0]).start()
                return carry

            lax.fori_loop(0, rpt_b, zstep, 0)

            def zdrain(b, carry):
                pltpu.make_async_copy(
                    zbuf, agg.at[pl.ds(sid * rpt + b * 16, 16)],
                    sem_s[0]).wait()
                return carry

            lax.fori_loop(0, rpt_b, zdrain, 0)
        else:
            pltpu.sync_copy(
                prev.at[pl.ds(cid * n_pad + sid * rpt, rpt)],
                agg.at[pl.ds(sid * rpt, rpt)],
            )
        plsc.subcore_barrier()

        def group(gi, carry):
            for b in range(KS):
                i = gi * KS + b
                pltpu.make_async_copy(
                    upd.at[pl.ds(ebase + i * CH, CH)], rows.at[b],
                    sem_r[b]).start()
            for b in range(KS):
                i = gi * KS + b
                pltpu.make_async_copy(
                    upd.at[pl.ds(ebase + i * CH, CH)], rows.at[b],
                    sem_r[b]).wait()
                pltpu.make_async_copy(
                    rows.at[b], agg.at[idx_r.at[i]], sem_s[b]).start(add=True)
            for b in range(KS):
                i = gi * KS + b
                pltpu.make_async_copy(
                    rows.at[b], agg.at[idx_r.at[i]], sem_s[b]).wait()
            return carry

        lax.fori_loop(0, ng, group, 0)
        for b in range(rem):
            i = ng * KS + b
            pltpu.make_async_copy(
                upd.at[pl.ds(ebase + i * CH, CH)], rows.at[b],
                sem_r[b]).start()
        for b in range(rem):
            i = ng * KS + b
            pltpu.make_async_copy(
                upd.at[pl.ds(ebase + i * CH, CH)], rows.at[b],
                sem_r[b]).wait()
            pltpu.make_async_copy(
                rows.at[b], agg.at[idx_r.at[i]], sem_s[b]).start(add=True)
        for b in range(rem):
            i = ng * KS + b
            pltpu.make_async_copy(
                rows.at[b], agg.at[idx_r.at[i]], sem_s[b]).wait()
        plsc.subcore_barrier()

        pltpu.sync_copy(
            agg.at[pl.ds(sid * rpt, rpt)],
            out.at[pl.ds(cid * n_pad + sid * rpt, rpt)],
        )

    if init is None:
        return k(updated_edges, rcv3d), n_pad
    return k(updated_edges, rcv3d, init), n_pad


def _edge_mlp_part(g_p, ef, We1e, We2, be2, eg, eb, off_blk, s_p, bpw,
                   upd_in, new_in):
    E, D = ef.shape
    BE = 2000
    nblk = g_p.shape[0] // BE
    assert nblk == NW * s_p

    def compute(g_blk, e_blk, w1, w2, b2, gm, bt):
        h = g_blk + jnp.dot(e_blk, w1, preferred_element_type=jnp.float32)
        h = jnp.maximum(h, 0.0)
        u = jnp.dot(h, w2, preferred_element_type=jnp.float32) + b2
        mu = jnp.mean(u, axis=-1, keepdims=True)
        var = jnp.mean((u - mu) ** 2, axis=-1, keepdims=True)
        return (u - mu) * jax.lax.rsqrt(var + EPS) * gm + bt

    part_map = lambda i, s=s_p, o=off_blk: (i // s * bpw + o + i % s, 0)
    own_map = lambda i: (i, 0)
    fixed_map = lambda i: (0, 0)
    small_specs = [
        pl.BlockSpec((D, D), fixed_map),
        pl.BlockSpec((D, D), fixed_map),
        pl.BlockSpec((1, D), fixed_map),
        pl.BlockSpec((1, D), fixed_map),
        pl.BlockSpec((1, D), fixed_map),
    ]
    n_alias = (upd_in is not None) + (new_in is not None)

    def body(*refs):
        g_r, ef_r, w1_r, w2_r, b2_r, gm_r, bt_r, upd_r, new_r = refs[n_alias:]
        e = ef_r[...]
        ln = compute(g_r[...], e, w1_r[...], w2_r[...], b2_r[...],
                     gm_r[...], bt_r[...])
        upd_r[...] = ln
        new_r[...] = ln + e

    alias_args, aliases = [], {}
    if upd_in is not None:
        aliases[len(alias_args)] = 0
        alias_args.append(upd_in)
    if new_in is not None:
        aliases[len(alias_args)] = 1
        alias_args.append(new_in)

    f = pl.pallas_call(
        body,
        grid=(nblk,),
        in_specs=[pl.BlockSpec(memory_space=pl.ANY)] * n_alias
                 + [pl.BlockSpec((BE, D), own_map),
                    pl.BlockSpec((BE, D), part_map)] + small_specs,
        out_specs=[pl.BlockSpec((BE, D), part_map),
                   pl.BlockSpec((BE, D), part_map)],
        out_shape=[
            jax.ShapeDtypeStruct((E, D), jnp.float32),
            jax.ShapeDtypeStruct((E, D), jnp.float32),
        ],
        input_output_aliases=aliases,
    )
    return f(*alias_args, g_p, ef, We1e, We2, be2.reshape(1, D),
             eg.reshape(1, D), eb.reshape(1, D))


def _node_mlp(nf, p0, p1, Wn1, bn1, Wn2, bn2, ng, nb):
    N, D = nf.shape
    BN = 1000
    grid = (N // BN,)

    def body(nf_r, p0_r, p1_r, w1_r, b1_r, w2_r, b2_r, g_r, b_r, out_r):
        w1 = w1_r[...]
        x = nf_r[...]
        agg = p0_r[...] + p1_r[...]
        h = jnp.dot(x, w1[0:D], preferred_element_type=jnp.float32)
        h += jnp.dot(agg, w1[D:2 * D], preferred_element_type=jnp.float32)
        h = jnp.maximum(h + b1_r[...], 0.0)
        u = jnp.dot(h, w2_r[...], preferred_element_type=jnp.float32) + b2_r[...]
        mu = jnp.mean(u, axis=-1, keepdims=True)
        var = jnp.mean((u - mu) ** 2, axis=-1, keepdims=True)
        ln = (u - mu) * jax.lax.rsqrt(var + EPS) * g_r[...] + b_r[...]
        out_r[...] = ln + x

    f = pl.pallas_call(
        body,
        grid=grid,
        in_specs=[
            pl.BlockSpec((BN, D), lambda i: (i, 0)),
            pl.BlockSpec((BN, D), lambda i: (i, 0)),
            pl.BlockSpec((BN, D), lambda i: (i, 0)),
            pl.BlockSpec((2 * D, D), lambda i: (0, 0)),
            pl.BlockSpec((1, D), lambda i: (0, 0)),
            pl.BlockSpec((D, D), lambda i: (0, 0)),
            pl.BlockSpec((1, D), lambda i: (0, 0)),
            pl.BlockSpec((1, D), lambda i: (0, 0)),
            pl.BlockSpec((1, D), lambda i: (0, 0)),
        ],
        out_specs=pl.BlockSpec((BN, D), lambda i: (i, 0)),
        out_shape=jax.ShapeDtypeStruct((N, D), jnp.float32),
    )
    return f(nf, p0, p1, Wn1, bn1.reshape(1, D), Wn2, bn2.reshape(1, D),
             ng.reshape(1, D), nb.reshape(1, D))


def kernel(node_features, edge_features, senders, receivers,
           We1, be1, We2, be2, eg, eb,
           Wn1, bn1, Wn2, bn2, ng, nb):
    N, D = node_features.shape
    E = edge_features.shape[0]
    chunks_full = E // (NW * CH)
    bpw = chunks_full * CH // 2000
    cpb = chunks_full // bpw
    snd3 = senders.reshape(NW, chunks_full, CH)
    rcv3 = receivers.reshape(NW, chunks_full, CH)
    ps, pr = _premult(node_features, We1, be1)
    We1e = We1[2 * D:3 * D]

    g0 = _sc_gather(ps, pr, snd3[:, :cpb], rcv3[:, :cpb])
    g1 = _sc_gather(ps, pr, snd3[:, cpb:3 * cpb], rcv3[:, cpb:3 * cpb])
    g2 = _sc_gather(ps, pr, snd3[:, 3 * cpb:], rcv3[:, 3 * cpb:])
    upd01, new01 = _edge_mlp_part(g0, edge_features, We1e, We2, be2, eg, eb,
                                  0, 1, bpw, None, None)
    upd01, new012 = _edge_mlp_part(g1, edge_features, We1e, We2, be2, eg, eb,
                                   1, 2, bpw, upd01, new01)
    upd2, new_edges = _edge_mlp_part(g2, edge_features, We1e, We2, be2, eg, eb,
                                     3, 2, bpw, None, new012)
    parts_a, n_pad = _sc_scatter(upd01, rcv3[:, :3 * cpb], N,
                                 chunks_full, 0, None)
    parts_c, _ = _sc_scatter(upd2, rcv3[:, 3 * cpb:], N,
                             chunks_full, 3 * cpb, parts_a)
    new_nodes = _node_mlp(node_features, parts_c[:N],
                          parts_c[n_pad:n_pad + N],
                          Wn1, bn1, Wn2, bn2, ng, nb)
    return (new_nodes, new_edges)
